# baseline (speedup 1.0000x reference)
import functools
import math

import jax
import jax.numpy as jnp
from jax import lax
from jax.experimental import pallas as pl
from jax.experimental.pallas import tpu as pltpu
from jax.experimental.pallas import tpu_sc as plsc

D = 128
PAD = 5000
SCALE = math.sqrt(float(D))
NC, NS, L = 2, 16, 16
NW = NC * NS
C = 128


@functools.lru_cache(maxsize=None)
def _build(N):
    rows_per_w = N // NW
    n_chunks = rows_per_w // C
    mesh = plsc.VectorSubcoreMesh(core_axis_name="c", subcore_axis_name="s")

    @functools.partial(
        pl.kernel,
        out_type=jax.ShapeDtypeStruct((N, D), jnp.float32),
        mesh=mesh,
        scratch_types=[
            pltpu.VMEM((C,), jnp.int32),
            pltpu.VMEM((C,), jnp.int32),
            pltpu.VMEM((C, D), jnp.float32),
            pltpu.VMEM((C, D), jnp.float32),
            pltpu.SemaphoreType.DMA,
        ],
    )
    def k(x_hbm, msk_hbm, idx_hbm, pe_hbm, out_hbm, idx_v, msk_v, x_v, rows_v, sem):
        wid = lax.axis_index("s") * NC + lax.axis_index("c")
        base = wid * rows_per_w

        def chunk(ci, carry):
            off = base + ci * C
            pltpu.sync_copy(idx_hbm.at[pl.ds(off, C)], idx_v)
            pltpu.sync_copy(msk_hbm.at[pl.ds(off, C)], msk_v)

            def fix(j, carry):
                s = pl.ds(j * L, L)
                iv = jnp.minimum(idx_v[s], PAD)
                idx_v[s] = jnp.where(msk_v[s] != 0, PAD, iv)
                return carry

            lax.fori_loop(0, C // L, fix, 0)
            gather = pltpu.async_copy(pe_hbm.at[idx_v], rows_v, sem)
            pltpu.sync_copy(x_hbm.at[pl.ds(off, C)], x_v)
            gather.wait()

            def fma(r, carry):
                for cb in range(D // L):
                    s = pl.ds(cb * L, L)
                    rows_v[r, s] = SCALE * x_v[r, s] + rows_v[r, s]
                return carry

            lax.fori_loop(0, C, fma, 0)
            pltpu.sync_copy(rows_v, out_hbm.at[pl.ds(off, C)])
            return carry

        lax.fori_loop(0, n_chunks, chunk, 0)

    return k


def kernel(x, mask, indices, pe):
    B, S, Dm = x.shape
    N = B * S
    x2 = x.reshape(N, Dm)
    msk = mask.reshape(N).astype(jnp.int32)
    idx = indices.reshape(N).astype(jnp.int32)
    out = _build(N)(x2, msk, idx, pe)
    return out.reshape(B, S, Dm)
